# baseline (speedup 1.0000x reference)
import numpy as np
import jax
import jax.numpy as jnp
from jax import lax
from jax.experimental import pallas as pl
from jax.experimental.pallas import tpu as pltpu
from jax.experimental.pallas import tpu_sc as plsc

_G = 7
_NSEL = _G * _G

_IDX = np.array([
    121, 35, 130, 148, 45, 176, 179, 139, 188, 99, 144, 152, 189, 31,
    112, 85, 63, 117, 174, 114, 82, 65, 7, 4, 101, 102, 78, 163, 157,
    183, 29, 177, 108, 83, 129, 44, 16, 58, 123, 37, 111, 19, 61, 2,
    142, 34, 156, 5, 90,
], dtype=np.int32)

_NW = 32
_H = 8


def _sc_body(pt_hbm, out_hbm, slab0, slab1, band0, band1, sem_g0, sem_g1,
             sem_s):
    B, C, p, _, N = pt_hbm.shape
    spw = (B * C) // _NW
    wid = lax.axis_index("s") * 2 + lax.axis_index("c")
    base = wid * spw
    lanes = jax.lax.broadcasted_iota(jnp.int32, (p,), 0)

    def gather(s, half, slab, sem):
        b = s // C
        c = s % C
        return pltpu.async_copy(pt_hbm.at[b, c, pl.ds(half * _H, _H)],
                                slab, sem)

    def compute(slab, band, pi0):
        @plsc.parallel_loop(0, _H, 1, unroll=4)
        def one_pi(pil):
            piv = jnp.full((p,), pil, jnp.int32)
            for gi in range(_G):
                row = gi * p + pi0 + pil
                for gj in range(_G):
                    n = int(_IDX[gi * _G + gj])
                    vec = plsc.load_gather(
                        slab, [piv, lanes, jnp.full((p,), n, jnp.int32)])
                    band[row, pl.ds(gj * p, p)] = vec

    def scatter(s, band):
        b = s // C
        c = s % C
        pltpu.async_copy(band, out_hbm.at[b, c], sem_s)

    def drain_scatters(k):
        for band in (band0, band1)[:k]:
            pltpu.make_async_copy(band, out_hbm.at[0, 0], sem_s).wait()

    def body(j, carry):
        sa = base + 2 * j
        sb = sa + 1

        @pl.when(j > 0)
        def _():
            drain_scatters(2)

        g0 = gather(sa, 0, slab0, sem_g0)
        g1 = gather(sa, 1, slab1, sem_g1)
        g0.wait()
        compute(slab0, band0, 0)
        g2 = gather(sb, 0, slab0, sem_g0)
        g1.wait()
        compute(slab1, band0, _H)
        scatter(sa, band0)
        g3 = gather(sb, 1, slab1, sem_g1)
        g2.wait()
        compute(slab0, band1, 0)
        g3.wait()
        compute(slab1, band1, _H)
        scatter(sb, band1)
        return carry

    lax.fori_loop(0, spw // 2, body, 0)
    drain_scatters(2)


def kernel(patches):
    B, N, C, p, _ = patches.shape
    pt = jnp.transpose(patches, (0, 2, 3, 4, 1))
    mesh = plsc.VectorSubcoreMesh(core_axis_name="c", subcore_axis_name="s")
    run = pl.kernel(
        _sc_body,
        mesh=mesh,
        compiler_params=pltpu.CompilerParams(use_tc_tiling_on_sc=True,
                                             needs_layout_passes=False),
        out_type=jax.ShapeDtypeStruct((B, C, _G * p, _G * p), jnp.float32),
        scratch_types=[
            pltpu.VMEM((_H, p, N), jnp.float32),
            pltpu.VMEM((_H, p, N), jnp.float32),
            pltpu.VMEM((_G * p, _G * p), jnp.float32),
            pltpu.VMEM((_G * p, _G * p), jnp.float32),
            pltpu.SemaphoreType.DMA,
            pltpu.SemaphoreType.DMA,
            pltpu.SemaphoreType.DMA,
        ],
    )
    return run(pt)


# speedup vs baseline: 1.4112x; 1.4112x over previous
import numpy as np
import jax
import jax.numpy as jnp
from jax import lax
from jax.experimental import pallas as pl
from jax.experimental.pallas import tpu as pltpu
from jax.experimental.pallas import tpu_sc as plsc

_G = 7
_NSEL = _G * _G

_IDX = np.array([
    121, 35, 130, 148, 45, 176, 179, 139, 188, 99, 144, 152, 189, 31,
    112, 85, 63, 117, 174, 114, 82, 65, 7, 4, 101, 102, 78, 163, 157,
    183, 29, 177, 108, 83, 129, 44, 16, 58, 123, 37, 111, 19, 61, 2,
    142, 34, 156, 5, 90,
], dtype=np.int32)

_NW = 32
_H = 8


def _sc_body(pt_hbm, out_hbm, slab0, slab1, band0, band1, sem_g0, sem_g1,
             sem_s):
    B, C, p, _, N = pt_hbm.shape
    spw = (B * C) // _NW
    wid = lax.axis_index("s") * 2 + lax.axis_index("c")
    base = wid * spw
    lanes = jax.lax.broadcasted_iota(jnp.int32, (p,), 0)

    def gather(s, half, slab, sem):
        b = s // C
        c = s % C
        return pltpu.async_copy(pt_hbm.at[b, c, pl.ds(half * _H, _H)],
                                slab, sem)

    def compute(slab, band, pi0):
        @plsc.parallel_loop(0, _H, 1, unroll=2)
        def one_pi(pil):
            piv = jnp.full((p,), pil, jnp.int32)
            for gi in range(_G):
                row = gi * p + pi0 + pil
                for gj in range(_G):
                    n = int(_IDX[gi * _G + gj])
                    vec = plsc.load_gather(
                        slab, [piv, lanes, jnp.full((p,), n, jnp.int32)])
                    band[row, pl.ds(gj * p, p)] = vec

    def scatter(s, band):
        b = s // C
        c = s % C
        pltpu.async_copy(band, out_hbm.at[b, c], sem_s)

    def drain_scatters(k):
        for band in (band0, band1)[:k]:
            pltpu.make_async_copy(band, out_hbm.at[0, 0], sem_s).wait()

    def drain_gather(slab, sem):
        pltpu.make_async_copy(pt_hbm.at[0, 0, pl.ds(0, _H)], slab, sem).wait()

    smax = B * C - 1

    def body(j, carry):
        sa = base + 2 * j
        sb = sa + 1
        sn = jnp.minimum(sa + 2, smax)

        @pl.when(j > 0)
        def _():
            drain_scatters(2)

        drain_gather(slab0, sem_g0)
        compute(slab0, band0, 0)
        gather(sb, 0, slab0, sem_g0)
        drain_gather(slab1, sem_g1)
        compute(slab1, band0, _H)
        scatter(sa, band0)
        gather(sb, 1, slab1, sem_g1)
        drain_gather(slab0, sem_g0)
        compute(slab0, band1, 0)
        gather(sn, 0, slab0, sem_g0)
        drain_gather(slab1, sem_g1)
        compute(slab1, band1, _H)
        scatter(sb, band1)
        gather(sn, 1, slab1, sem_g1)
        return carry

    gather(base, 0, slab0, sem_g0)
    gather(base, 1, slab1, sem_g1)
    lax.fori_loop(0, spw // 2, body, 0)
    drain_gather(slab0, sem_g0)
    drain_gather(slab1, sem_g1)
    drain_scatters(2)


def kernel(patches):
    B, N, C, p, _ = patches.shape
    pt = jnp.transpose(patches, (0, 2, 3, 4, 1))
    mesh = plsc.VectorSubcoreMesh(core_axis_name="c", subcore_axis_name="s")
    run = pl.kernel(
        _sc_body,
        mesh=mesh,
        compiler_params=pltpu.CompilerParams(use_tc_tiling_on_sc=True,
                                             needs_layout_passes=False),
        out_type=jax.ShapeDtypeStruct((B, C, _G * p, _G * p), jnp.float32),
        scratch_types=[
            pltpu.VMEM((_H, p, N), jnp.float32),
            pltpu.VMEM((_H, p, N), jnp.float32),
            pltpu.VMEM((_G * p, _G * p), jnp.float32),
            pltpu.VMEM((_G * p, _G * p), jnp.float32),
            pltpu.SemaphoreType.DMA,
            pltpu.SemaphoreType.DMA,
            pltpu.SemaphoreType.DMA,
        ],
    )
    return run(pt)
